# baseline (speedup 1.0000x reference)
import functools

import jax
import jax.numpy as jnp
from jax import lax
from jax.experimental import pallas as pl
from jax.experimental.pallas import tpu as pltpu
from jax.experimental.pallas import tpu_sc as plsc

_N = 10000
_E = 320000
_H = 128
_HH = 64
_OUT = 16
_G = 64
_V = 500
_VPAD = 512

_NC = 2
_NS = 16

_NPAD = 10240
_ROWS_S = _NPAD // _NS
_EPT = _E // _NS
_CW = 128
_NB = 2
_EPB = _EPT // _NB
_CHB = 80
_CHBG = _CHB + 2
_XCH = _ROWS_S // _CW

_BLK = 640
_NBLK = _NPAD // _BLK

_MESH = plsc.VectorSubcoreMesh(core_axis_name="c", subcore_axis_name="s")


def _run_edges(s, agg_sh, h_sh, srcv, dstv, rows, rsem, wsem,
               src_hbm, dst_hbm):
    def fire_gather(chunk, b):
        pltpu.async_copy(h_sh.at[srcv.at[pl.ds(chunk * _CW, _CW)]],
                         rows[b], rsem[b])

    def wait_gather(b):
        pltpu.make_async_copy(h_sh.at[srcv.at[pl.ds(0, _CW)]],
                              rows[b], rsem[b]).wait()

    def fire_scatter(chunk, b):
        pltpu.async_copy(rows[b], agg_sh.at[dstv.at[chunk]], wsem[b],
                         add=True)

    def wait_scatter(b):
        pltpu.make_async_copy(rows[b], agg_sh.at[dstv.at[0]], wsem[b]).wait()

    for batch in range(_NB):
        if batch > 0:
            pltpu.sync_copy(src_hbm.at[s].at[batch], srcv)
            pltpu.sync_copy(dst_hbm.at[s].at[batch], dstv)
        fire_gather(0, 0)
        fire_gather(1, 1)
        wait_gather(0)
        fire_scatter(0, 0)
        fire_gather(2, 2)
        wait_gather(1)
        fire_scatter(1, 1)
        wait_scatter(0)
        fire_gather(3, 0)

        def body(i, carry):
            for u in range(6):
                j = 6 * i + 2 + u
                b = (2 + u) % 3
                wait_gather(b)
                fire_scatter(j, b)
                wait_scatter((2 + u + 2) % 3)
                fire_gather(j + 2, (2 + u + 2) % 3)
            return carry

        lax.fori_loop(0, (_CHB - 2) // 6, body, 0)
        wait_gather(2)
        wait_gather(0)
        wait_scatter(1)


_EDGE_SCRATCH = [
    pltpu.VMEM_SHARED((_NPAD, _HH), jnp.float32),
    pltpu.VMEM_SHARED((_NPAD, _HH), jnp.float32),
    pltpu.VMEM((_CHBG * _CW,), jnp.int32),
    pltpu.VMEM((_CHB, _CW), jnp.int32),
    pltpu.VMEM((_CW, _HH), jnp.float32),
    pltpu.VMEM((_CW, _HH), jnp.float32),
    pltpu.VMEM((_CW, _HH), jnp.float32),
    pltpu.SemaphoreType.DMA,
    pltpu.SemaphoreType.DMA,
    pltpu.SemaphoreType.DMA,
    pltpu.SemaphoreType.DMA,
    pltpu.SemaphoreType.DMA,
    pltpu.SemaphoreType.DMA,
]


@functools.partial(
    pl.kernel,
    out_type=jax.ShapeDtypeStruct((_NC, _NPAD, _HH), jnp.float32),
    mesh=_MESH,
    scratch_types=_EDGE_SCRATCH + [pltpu.VMEM((_XCH, _CW), jnp.int32)],
    compiler_params=pltpu.CompilerParams(use_tc_tiling_on_sc=False),
)
def _sc_embed_agg(tab_hbm, xi_hbm, src_hbm, dst_hbm, out_hbm,
                  agg_sh, h_sh, srcv, dstv, rows0, rows1, rows2,
                  rs0, rs1, rs2, ws0, ws1, ws2, idxv):
    c = lax.axis_index("c")
    s = lax.axis_index("s")
    rows = (rows0, rows1, rows2)
    rsem = (rs0, rs1, rs2)
    wsem = (ws0, ws1, ws2)
    base = s * _ROWS_S
    pltpu.async_copy(src_hbm.at[s].at[0], srcv, ws0)
    pltpu.async_copy(dst_hbm.at[s].at[0], dstv, ws1)
    pltpu.sync_copy(xi_hbm.at[s], idxv)
    for j in range(_XCH):
        pltpu.async_copy(tab_hbm.at[c].at[idxv.at[j]], rows0, rs0).wait()
        slj = pl.ds(base + j * _CW, _CW)
        pltpu.sync_copy(rows0, h_sh.at[slj])
        pltpu.sync_copy(rows0, agg_sh.at[slj])
    plsc.subcore_barrier()
    pltpu.make_async_copy(src_hbm.at[s].at[0], srcv, ws0).wait()
    pltpu.make_async_copy(dst_hbm.at[s].at[0], dstv, ws1).wait()
    _run_edges(s, agg_sh, h_sh, srcv, dstv, rows, rsem, wsem,
               src_hbm, dst_hbm)
    plsc.subcore_barrier()
    sl = pl.ds(base, _ROWS_S)
    pltpu.sync_copy(agg_sh.at[sl], out_hbm.at[c].at[sl])


@functools.partial(
    pl.kernel,
    out_type=jax.ShapeDtypeStruct((_NC, _NPAD, _HH), jnp.float32),
    mesh=_MESH,
    scratch_types=_EDGE_SCRATCH,
    compiler_params=pltpu.CompilerParams(use_tc_tiling_on_sc=False),
)
def _sc_edge_agg(h_hbm, src_hbm, dst_hbm, out_hbm,
                 agg_sh, h_sh, srcv, dstv, rows0, rows1, rows2,
                 rs0, rs1, rs2, ws0, ws1, ws2):
    c = lax.axis_index("c")
    s = lax.axis_index("s")
    rows = (rows0, rows1, rows2)
    rsem = (rs0, rs1, rs2)
    wsem = (ws0, ws1, ws2)
    h_c = h_hbm.at[c]
    sl = pl.ds(s * _ROWS_S, _ROWS_S)
    pltpu.async_copy(h_c.at[sl], agg_sh.at[sl], ws0)
    pltpu.async_copy(h_c.at[sl], h_sh.at[sl], ws1)
    pltpu.async_copy(src_hbm.at[s].at[0], srcv, rs0)
    pltpu.async_copy(dst_hbm.at[s].at[0], dstv, rs1)
    pltpu.make_async_copy(h_c.at[sl], agg_sh.at[sl], ws0).wait()
    pltpu.make_async_copy(h_c.at[sl], h_sh.at[sl], ws1).wait()
    plsc.subcore_barrier()
    pltpu.make_async_copy(src_hbm.at[s].at[0], srcv, rs0).wait()
    pltpu.make_async_copy(dst_hbm.at[s].at[0], dstv, rs1).wait()
    _run_edges(s, agg_sh, h_sh, srcv, dstv, rows, rsem, wsem,
               src_hbm, dst_hbm)
    plsc.subcore_barrier()
    pltpu.sync_copy(agg_sh.at[sl], out_hbm.at[c].at[sl])


def _table_body(emb_ref, w1_ref, b1_ref, w2_ref, b2_ref, out_ref):
    t = jnp.dot(emb_ref[...], w1_ref[...], preferred_element_type=jnp.float32)
    t = jnp.maximum(t + b1_ref[...], 0.0)
    tab = jnp.dot(t, w2_ref[...], preferred_element_type=jnp.float32) + b2_ref[...]
    out_ref[0, :, :] = tab[:, :_HH]
    out_ref[1, :, :] = tab[:, _HH:]


def _tc_table(emb_p, w1, b1, w2, b2):
    return pl.pallas_call(
        _table_body,
        out_shape=jax.ShapeDtypeStruct((_NC, _VPAD, _HH), jnp.float32),
    )(emb_p, w1, b1, w2, b2)


def _mlp_bn_body(a0_ref, a1_ref, w1_ref, b1_ref, w2_ref, b2_ref,
                 g_ref, beta_ref, out_ref, v_scr, stats_scr):
    p = pl.program_id(0)
    k = pl.program_id(1)

    @pl.when(p == 0)
    def _():
        t = jnp.concatenate([a0_ref[0], a1_ref[0]], axis=1)
        u = jnp.dot(t, w1_ref[...], preferred_element_type=jnp.float32)
        u = jnp.maximum(u + b1_ref[...], 0.0)
        v = (jnp.dot(u, w2_ref[...], preferred_element_type=jnp.float32)
             + b2_ref[...])
        v_scr[pl.ds(k * _BLK, _BLK), :] = v
        rows = lax.broadcasted_iota(jnp.int32, (_BLK, 1), 0) + k * _BLK
        vm = jnp.where(rows < _N, v, 0.0)
        part = jnp.concatenate(
            [jnp.sum(vm, axis=0, keepdims=True),
             jnp.sum(vm * vm, axis=0, keepdims=True),
             jnp.zeros((6, _H), jnp.float32)], axis=0)

        @pl.when(k == 0)
        def _():
            stats_scr[...] = part

        @pl.when(k > 0)
        def _():
            stats_scr[...] += part

    @pl.when(p == 1)
    def _():
        st = stats_scr[...]
        mu = st[0:1, :] / float(_N)
        var = st[1:2, :] / float(_N) - mu * mu
        inv = lax.rsqrt(var + 1e-5)
        hn = jnp.maximum(
            (v_scr[pl.ds(k * _BLK, _BLK), :] - mu) * inv * g_ref[...]
            + beta_ref[...], 0.0)
        out_ref[0, :, :] = hn[:, :_HH]
        out_ref[1, :, :] = hn[:, _HH:]


def _tc_mlp_bn(agg2, w1, b1, w2, b2, g, beta):
    return pl.pallas_call(
        _mlp_bn_body,
        grid=(2, _NBLK),
        in_specs=[
            pl.BlockSpec((1, _BLK, _HH), lambda p, k: (0, k, 0)),
            pl.BlockSpec((1, _BLK, _HH), lambda p, k: (1, k, 0)),
            pl.BlockSpec((_H, _H), lambda p, k: (0, 0)),
            pl.BlockSpec((1, _H), lambda p, k: (0, 0)),
            pl.BlockSpec((_H, _H), lambda p, k: (0, 0)),
            pl.BlockSpec((1, _H), lambda p, k: (0, 0)),
            pl.BlockSpec((1, _H), lambda p, k: (0, 0)),
            pl.BlockSpec((1, _H), lambda p, k: (0, 0)),
        ],
        out_specs=pl.BlockSpec((_NC, _BLK, _HH), lambda p, k: (0, k * p, 0)),
        out_shape=jax.ShapeDtypeStruct((_NC, _NPAD, _HH), jnp.float32),
        scratch_shapes=[
            pltpu.VMEM((_NPAD, _H), jnp.float32),
            pltpu.VMEM((8, _H), jnp.float32),
        ],
        compiler_params=pltpu.CompilerParams(
            dimension_semantics=("arbitrary", "arbitrary")),
    )(agg2, agg2, w1, b1, w2, b2, g, beta)


def _pool_final_body(h0_ref, h1_ref, b_ref, wf1_ref, bf1_ref,
                     wf2_ref, bf2_ref, out_ref, pool_scr):
    k = pl.program_id(0)
    h = jnp.concatenate([h0_ref[0], h1_ref[0]], axis=1)
    b = b_ref[0, 0, :]
    gids = lax.broadcasted_iota(jnp.int32, (_G, _BLK), 0)
    oh = (gids == b[None, :]).astype(jnp.float32)
    part = jnp.dot(oh, h, preferred_element_type=jnp.float32)

    @pl.when(k == 0)
    def _():
        pool_scr[...] = part

    @pl.when(k > 0)
    def _():
        pool_scr[...] += part

    @pl.when(k == _NBLK - 1)
    def _():
        r = jnp.dot(pool_scr[...], wf1_ref[...],
                    preferred_element_type=jnp.float32)
        r = jnp.maximum(r + bf1_ref[...], 0.0)
        out_ref[...] = (
            jnp.dot(r, wf2_ref[...], preferred_element_type=jnp.float32)
            + bf2_ref[...])


def _tc_pool_final(h2, batch3, wf1, bf1, wf2, bf2):
    return pl.pallas_call(
        _pool_final_body,
        grid=(_NBLK,),
        in_specs=[
            pl.BlockSpec((1, _BLK, _HH), lambda k: (0, k, 0)),
            pl.BlockSpec((1, _BLK, _HH), lambda k: (1, k, 0)),
            pl.BlockSpec((1, 1, _BLK), lambda k: (k, 0, 0)),
            pl.BlockSpec((_H, _H), lambda k: (0, 0)),
            pl.BlockSpec((1, _H), lambda k: (0, 0)),
            pl.BlockSpec((_H, _OUT), lambda k: (0, 0)),
            pl.BlockSpec((1, _OUT), lambda k: (0, 0)),
        ],
        out_specs=pl.BlockSpec((_G, _OUT), lambda k: (0, 0)),
        out_shape=jax.ShapeDtypeStruct((_G, _OUT), jnp.float32),
        scratch_shapes=[pltpu.VMEM((_G, _H), jnp.float32)],
        compiler_params=pltpu.CompilerParams(
            dimension_semantics=("arbitrary",)),
    )(h2, h2, batch3, wf1, bf1, wf2, bf2)


def _row(b):
    return b.reshape(1, -1)


def kernel(x, edge_index, batch, params):
    p = params
    src, dst = edge_index[0], edge_index[1]

    emb_p = jnp.pad(p['emb'], ((0, _VPAD - _V), (0, 0)))
    xi = jnp.pad(x, (0, _NPAD - _N)).reshape(_NS, _XCH, _CW)
    srcf = jnp.pad(src.reshape(_NS, _NB, _EPB),
                   ((0, 0), (0, 0), (0, _CHBG * _CW - _EPB)))
    dst3 = jnp.pad(
        dst.reshape(_NS, _NB, _EPB), ((0, 0), (0, 0), (0, _CHB * _CW - _EPB)),
        constant_values=_N,
    ).reshape(_NS, _NB, _CHB, _CW)
    batch3 = jnp.pad(batch, (0, _NPAD - _N), constant_values=_G).reshape(
        _NBLK, 1, _BLK)
    tab2 = _tc_table(emb_p, p['Wi1'], _row(p['bi1']), p['Wi2'], _row(p['bi2']))
    h2 = None
    for i, cp in enumerate(p['convs']):
        if i == 0:
            agg2 = _sc_embed_agg(tab2, xi, srcf, dst3)
        else:
            agg2 = _sc_edge_agg(h2, srcf, dst3)
        h2 = _tc_mlp_bn(agg2, cp['W1'], _row(cp['b1']), cp['W2'],
                        _row(cp['b2']), _row(cp['g']), _row(cp['beta']))
    return _tc_pool_final(h2, batch3, p['Wf1'], _row(p['bf1']),
                          p['Wf2'], _row(p['bf2']))


# speedup vs baseline: 1.0262x; 1.0262x over previous
import functools

import jax
import jax.numpy as jnp
from jax import lax
from jax.experimental import pallas as pl
from jax.experimental.pallas import tpu as pltpu
from jax.experimental.pallas import tpu_sc as plsc

_N = 10000
_E = 320000
_H = 128
_HH = 64
_OUT = 16
_G = 64
_V = 500
_VPAD = 512

_NC = 2
_NS = 16

_NPAD = 10240
_ROWS_S = _NPAD // _NS
_EPT = _E // _NS
_CW = 128
_NB = 2
_EPB = _EPT // _NB
_CHB = 80
_CHBG = _CHB + 2
_XCH = _ROWS_S // _CW

_BLK = 640
_NBLK = _NPAD // _BLK

_MESH = plsc.VectorSubcoreMesh(core_axis_name="c", subcore_axis_name="s")


def _run_edges(s, agg_sh, h_sh, srcv, dstv, rows, rsem, wsem,
               src_hbm, dst_hbm):
    def fire_gather(chunk, b):
        pltpu.async_copy(h_sh.at[srcv.at[pl.ds(chunk * _CW, _CW)]],
                         rows[b], rsem[b])

    def wait_gather(b):
        pltpu.make_async_copy(h_sh.at[srcv.at[pl.ds(0, _CW)]],
                              rows[b], rsem[b]).wait()

    def fire_scatter(chunk, b):
        pltpu.async_copy(rows[b], agg_sh.at[dstv.at[chunk]], wsem[b],
                         add=True)

    def wait_scatter(b):
        pltpu.make_async_copy(rows[b], agg_sh.at[dstv.at[0]], wsem[b]).wait()

    for batch in range(_NB):
        if batch > 0:
            pltpu.sync_copy(src_hbm.at[s].at[batch], srcv)
            pltpu.sync_copy(dst_hbm.at[s].at[batch], dstv)
        fire_gather(0, 0)
        fire_gather(1, 1)
        wait_gather(0)
        fire_scatter(0, 0)
        fire_gather(2, 2)
        wait_gather(1)
        fire_scatter(1, 1)
        wait_scatter(0)
        fire_gather(3, 0)

        def body(i, carry):
            for u in range(6):
                j = 6 * i + 2 + u
                b = (2 + u) % 3
                wait_gather(b)
                fire_scatter(j, b)
                wait_scatter((2 + u + 2) % 3)
                fire_gather(j + 2, (2 + u + 2) % 3)
            return carry

        lax.fori_loop(0, (_CHB - 2) // 6, body, 0)
        wait_gather(2)
        wait_gather(0)
        wait_scatter(1)


_EDGE_SCRATCH = [
    pltpu.VMEM_SHARED((_NPAD, _HH), jnp.float32),
    pltpu.VMEM_SHARED((_NPAD, _HH), jnp.float32),
    pltpu.VMEM((_CHBG * _CW,), jnp.int32),
    pltpu.VMEM((_CHB, _CW), jnp.int32),
    pltpu.VMEM((_CW, _HH), jnp.float32),
    pltpu.VMEM((_CW, _HH), jnp.float32),
    pltpu.VMEM((_CW, _HH), jnp.float32),
    pltpu.SemaphoreType.DMA,
    pltpu.SemaphoreType.DMA,
    pltpu.SemaphoreType.DMA,
    pltpu.SemaphoreType.DMA,
    pltpu.SemaphoreType.DMA,
    pltpu.SemaphoreType.DMA,
]


@functools.partial(
    pl.kernel,
    out_type=jax.ShapeDtypeStruct((_NC, _NPAD, _HH), jnp.float32),
    mesh=_MESH,
    scratch_types=_EDGE_SCRATCH + [pltpu.VMEM((_XCH, _CW), jnp.int32)],
    compiler_params=pltpu.CompilerParams(use_tc_tiling_on_sc=False),
)
def _sc_embed_agg(tab_hbm, xi_hbm, src_hbm, dst_hbm, out_hbm,
                  agg_sh, h_sh, srcv, dstv, rows0, rows1, rows2,
                  rs0, rs1, rs2, ws0, ws1, ws2, idxv):
    c = lax.axis_index("c")
    s = lax.axis_index("s")
    rows = (rows0, rows1, rows2)
    rsem = (rs0, rs1, rs2)
    wsem = (ws0, ws1, ws2)
    base = s * _ROWS_S
    pltpu.async_copy(src_hbm.at[s].at[0], srcv, ws0)
    pltpu.async_copy(dst_hbm.at[s].at[0], dstv, ws1)
    pltpu.sync_copy(xi_hbm.at[s], idxv)
    for j in range(_XCH):
        pltpu.async_copy(tab_hbm.at[c].at[idxv.at[j]], rows0, rs0).wait()
        slj = pl.ds(base + j * _CW, _CW)
        pltpu.sync_copy(rows0, h_sh.at[slj])
        pltpu.sync_copy(rows0, agg_sh.at[slj])
    plsc.subcore_barrier()
    pltpu.make_async_copy(src_hbm.at[s].at[0], srcv, ws0).wait()
    pltpu.make_async_copy(dst_hbm.at[s].at[0], dstv, ws1).wait()
    _run_edges(s, agg_sh, h_sh, srcv, dstv, rows, rsem, wsem,
               src_hbm, dst_hbm)
    plsc.subcore_barrier()
    sl = pl.ds(base, _ROWS_S)
    pltpu.sync_copy(agg_sh.at[sl], out_hbm.at[c].at[sl])


@functools.partial(
    pl.kernel,
    out_type=jax.ShapeDtypeStruct((_NC, _NPAD, _HH), jnp.float32),
    mesh=_MESH,
    scratch_types=_EDGE_SCRATCH,
    compiler_params=pltpu.CompilerParams(use_tc_tiling_on_sc=False),
)
def _sc_edge_agg(h_hbm, src_hbm, dst_hbm, out_hbm,
                 agg_sh, h_sh, srcv, dstv, rows0, rows1, rows2,
                 rs0, rs1, rs2, ws0, ws1, ws2):
    c = lax.axis_index("c")
    s = lax.axis_index("s")
    rows = (rows0, rows1, rows2)
    rsem = (rs0, rs1, rs2)
    wsem = (ws0, ws1, ws2)
    h_c = h_hbm.at[c]
    sl = pl.ds(s * _ROWS_S, _ROWS_S)
    pltpu.async_copy(h_c.at[sl], agg_sh.at[sl], ws0)
    pltpu.async_copy(h_c.at[sl], h_sh.at[sl], ws1)
    pltpu.async_copy(src_hbm.at[s].at[0], srcv, rs0)
    pltpu.async_copy(dst_hbm.at[s].at[0], dstv, rs1)
    pltpu.make_async_copy(h_c.at[sl], agg_sh.at[sl], ws0).wait()
    pltpu.make_async_copy(h_c.at[sl], h_sh.at[sl], ws1).wait()
    plsc.subcore_barrier()
    pltpu.make_async_copy(src_hbm.at[s].at[0], srcv, rs0).wait()
    pltpu.make_async_copy(dst_hbm.at[s].at[0], dstv, rs1).wait()
    _run_edges(s, agg_sh, h_sh, srcv, dstv, rows, rsem, wsem,
               src_hbm, dst_hbm)
    plsc.subcore_barrier()
    pltpu.sync_copy(agg_sh.at[sl], out_hbm.at[c].at[sl])


def _table_body(emb_ref, w1_ref, b1_ref, w2_ref, b2_ref, out_ref):
    t = jnp.dot(emb_ref[...], w1_ref[...], preferred_element_type=jnp.float32)
    t = jnp.maximum(t + b1_ref[...], 0.0)
    tab = jnp.dot(t, w2_ref[...], preferred_element_type=jnp.float32) + b2_ref[...]
    out_ref[0, :, :] = tab[:, :_HH]
    out_ref[1, :, :] = tab[:, _HH:]


def _tc_table(emb_p, w1, b1, w2, b2):
    return pl.pallas_call(
        _table_body,
        out_shape=jax.ShapeDtypeStruct((_NC, _VPAD, _HH), jnp.float32),
    )(emb_p, w1, b1, w2, b2)


def _mlp_bn_body(a0_ref, a1_ref, w1_ref, b1_ref, w2_ref, b2_ref,
                 g_ref, beta_ref, out_ref, v_scr, stats_scr):
    p = pl.program_id(0)
    k = pl.program_id(1)

    @pl.when(p == 0)
    def _():
        t = jnp.concatenate([a0_ref[0], a1_ref[0]], axis=1)
        u = jnp.dot(t, w1_ref[...], preferred_element_type=jnp.float32)
        u = jnp.maximum(u + b1_ref[...], 0.0)
        v = (jnp.dot(u, w2_ref[...], preferred_element_type=jnp.float32)
             + b2_ref[...])
        v_scr[pl.ds(k * _BLK, _BLK), :] = v
        rows = lax.broadcasted_iota(jnp.int32, (_BLK, 1), 0) + k * _BLK
        vm = jnp.where(rows < _N, v, 0.0)
        part = jnp.concatenate(
            [jnp.sum(vm, axis=0, keepdims=True),
             jnp.sum(vm * vm, axis=0, keepdims=True),
             jnp.zeros((6, _H), jnp.float32)], axis=0)

        @pl.when(k == 0)
        def _():
            stats_scr[...] = part

        @pl.when(k > 0)
        def _():
            stats_scr[...] += part

    @pl.when(p == 1)
    def _():
        st = stats_scr[...]
        mu = st[0:1, :] / float(_N)
        var = st[1:2, :] / float(_N) - mu * mu
        inv = lax.rsqrt(var + 1e-5)
        hn = jnp.maximum(
            (v_scr[pl.ds(k * _BLK, _BLK), :] - mu) * inv * g_ref[...]
            + beta_ref[...], 0.0)
        out_ref[0, :, :] = hn[:, :_HH]
        out_ref[1, :, :] = hn[:, _HH:]


def _tc_mlp_bn(agg2, w1, b1, w2, b2, g, beta):
    return pl.pallas_call(
        _mlp_bn_body,
        grid=(2, _NBLK),
        in_specs=[
            pl.BlockSpec((1, _BLK, _HH), lambda p, k: (0, k * (1 - p), 0)),
            pl.BlockSpec((1, _BLK, _HH), lambda p, k: (1, k * (1 - p), 0)),
            pl.BlockSpec((_H, _H), lambda p, k: (0, 0)),
            pl.BlockSpec((1, _H), lambda p, k: (0, 0)),
            pl.BlockSpec((_H, _H), lambda p, k: (0, 0)),
            pl.BlockSpec((1, _H), lambda p, k: (0, 0)),
            pl.BlockSpec((1, _H), lambda p, k: (0, 0)),
            pl.BlockSpec((1, _H), lambda p, k: (0, 0)),
        ],
        out_specs=pl.BlockSpec((_NC, _BLK, _HH), lambda p, k: (0, k * p, 0)),
        out_shape=jax.ShapeDtypeStruct((_NC, _NPAD, _HH), jnp.float32),
        scratch_shapes=[
            pltpu.VMEM((_NPAD, _H), jnp.float32),
            pltpu.VMEM((8, _H), jnp.float32),
        ],
        compiler_params=pltpu.CompilerParams(
            dimension_semantics=("arbitrary", "arbitrary")),
    )(agg2, agg2, w1, b1, w2, b2, g, beta)


def _pool_final_body(h0_ref, h1_ref, b_ref, wf1_ref, bf1_ref,
                     wf2_ref, bf2_ref, out_ref, pool_scr):
    k = pl.program_id(0)
    h = jnp.concatenate([h0_ref[0], h1_ref[0]], axis=1)
    b = b_ref[0, 0, :]
    gids = lax.broadcasted_iota(jnp.int32, (_G, _BLK), 0)
    oh = (gids == b[None, :]).astype(jnp.float32)
    part = jnp.dot(oh, h, preferred_element_type=jnp.float32)

    @pl.when(k == 0)
    def _():
        pool_scr[...] = part

    @pl.when(k > 0)
    def _():
        pool_scr[...] += part

    @pl.when(k == _NBLK - 1)
    def _():
        r = jnp.dot(pool_scr[...], wf1_ref[...],
                    preferred_element_type=jnp.float32)
        r = jnp.maximum(r + bf1_ref[...], 0.0)
        out_ref[...] = (
            jnp.dot(r, wf2_ref[...], preferred_element_type=jnp.float32)
            + bf2_ref[...])


def _tc_pool_final(h2, batch3, wf1, bf1, wf2, bf2):
    return pl.pallas_call(
        _pool_final_body,
        grid=(_NBLK,),
        in_specs=[
            pl.BlockSpec((1, _BLK, _HH), lambda k: (0, k, 0)),
            pl.BlockSpec((1, _BLK, _HH), lambda k: (1, k, 0)),
            pl.BlockSpec((1, 1, _BLK), lambda k: (k, 0, 0)),
            pl.BlockSpec((_H, _H), lambda k: (0, 0)),
            pl.BlockSpec((1, _H), lambda k: (0, 0)),
            pl.BlockSpec((_H, _OUT), lambda k: (0, 0)),
            pl.BlockSpec((1, _OUT), lambda k: (0, 0)),
        ],
        out_specs=pl.BlockSpec((_G, _OUT), lambda k: (0, 0)),
        out_shape=jax.ShapeDtypeStruct((_G, _OUT), jnp.float32),
        scratch_shapes=[pltpu.VMEM((_G, _H), jnp.float32)],
        compiler_params=pltpu.CompilerParams(
            dimension_semantics=("arbitrary",)),
    )(h2, h2, batch3, wf1, bf1, wf2, bf2)


def _row(b):
    return b.reshape(1, -1)


def kernel(x, edge_index, batch, params):
    p = params
    src, dst = edge_index[0], edge_index[1]

    emb_p = jnp.pad(p['emb'], ((0, _VPAD - _V), (0, 0)))
    xi = jnp.pad(x, (0, _NPAD - _N)).reshape(_NS, _XCH, _CW)
    srcf = jnp.pad(src.reshape(_NS, _NB, _EPB),
                   ((0, 0), (0, 0), (0, _CHBG * _CW - _EPB)))
    dst3 = jnp.pad(
        dst.reshape(_NS, _NB, _EPB), ((0, 0), (0, 0), (0, _CHB * _CW - _EPB)),
        constant_values=_N,
    ).reshape(_NS, _NB, _CHB, _CW)
    batch3 = jnp.pad(batch, (0, _NPAD - _N), constant_values=_G).reshape(
        _NBLK, 1, _BLK)
    tab2 = _tc_table(emb_p, p['Wi1'], _row(p['bi1']), p['Wi2'], _row(p['bi2']))
    h2 = None
    for i, cp in enumerate(p['convs']):
        if i == 0:
            agg2 = _sc_embed_agg(tab2, xi, srcf, dst3)
        else:
            agg2 = _sc_edge_agg(h2, srcf, dst3)
        h2 = _tc_mlp_bn(agg2, cp['W1'], _row(cp['b1']), cp['W2'],
                        _row(cp['b2']), _row(cp['g']), _row(cp['beta']))
    return _tc_pool_final(h2, batch3, p['Wf1'], _row(p['bf1']),
                          p['Wf2'], _row(p['bf2']))
